# baseline (speedup 1.0000x reference)
import jax
import jax.numpy as jnp
from jax import lax
from jax.experimental import pallas as pl
from jax.experimental.pallas import tpu as pltpu
from jax.experimental.pallas import tpu_sc as plsc

_B = 4096
_E = 100000
_D0, _D1, _D2 = 128, 64, 32
_NC, _NS, _L = 2, 16, 16
_NW = _NC * _NS
_BPW = _B // _NW


def _sc_body(idx_hbm, t0_hbm, t1_hbm, t2_hbm, out_hbm,
             idx_v, idx1_v, idx2_v, b0, b1, b2, sem0, sem1, sem2):
    wid = lax.axis_index("s") * _NC + lax.axis_index("c")
    base = wid * _BPW
    pltpu.sync_copy(idx_hbm.at[pl.ds(base, _BPW)], idx_v)
    c0 = pltpu.async_copy(t0_hbm.at[idx_v], b0, sem0)
    for j in range(_BPW // _L):
        v = idx_v[pl.ds(_L * j, _L)]
        idx1_v[pl.ds(_L * j, _L)] = v >> 1
        idx2_v[pl.ds(_L * j, _L)] = v >> 2
    c1 = pltpu.async_copy(t1_hbm.at[idx1_v], b1, sem1)
    c2 = pltpu.async_copy(t2_hbm.at[idx2_v], b2, sem2)
    c2.wait()
    c1.wait()
    c0.wait()
    third = jnp.float32(1.0 / 3.0)
    lane = lax.iota(jnp.int32, _L)

    def row_body(r, carry):
        rvec = jnp.full((_L,), r, jnp.int32)
        iv = plsc.load_gather(idx_v, [rvec])
        c1base = ((iv & 1) << 6) + lane
        c2base = ((iv & 3) << 5) + lane
        for j in range(_D0 // _L):
            v = b0[r, pl.ds(_L * j, _L)]
            if _L * j < _D1:
                v = v + plsc.load_gather(b1, [rvec, c1base + _L * j])
            if _L * j < _D2:
                v = v + plsc.load_gather(b2, [rvec, c2base + _L * j])
            b0[r, pl.ds(_L * j, _L)] = v * third
        return carry

    lax.fori_loop(0, _BPW, row_body, 0)
    pltpu.sync_copy(b0, out_hbm.at[pl.ds(base, _BPW)])


def kernel(indexes, table0, table1, table2):
    t1r = jnp.reshape(table1, (_E * _D1 // _D0, _D0))
    t2r = jnp.reshape(table2, (_E * _D2 // _D0, _D0))
    mesh = plsc.VectorSubcoreMesh(core_axis_name="c", subcore_axis_name="s")
    k = pl.kernel(
        _sc_body,
        out_type=jax.ShapeDtypeStruct((_B, _D0), jnp.float32),
        mesh=mesh,
        compiler_params=pltpu.CompilerParams(needs_layout_passes=False),
        scratch_types=[
            pltpu.VMEM((_BPW,), jnp.int32),
            pltpu.VMEM((_BPW,), jnp.int32),
            pltpu.VMEM((_BPW,), jnp.int32),
            pltpu.VMEM((_BPW, _D0), jnp.float32),
            pltpu.VMEM((_BPW, _D0), jnp.float32),
            pltpu.VMEM((_BPW, _D0), jnp.float32),
            pltpu.SemaphoreType.DMA,
            pltpu.SemaphoreType.DMA,
            pltpu.SemaphoreType.DMA,
        ],
    )
    return k(indexes.astype(jnp.int32), table0, t1r, t2r)


# speedup vs baseline: 1.4556x; 1.4556x over previous
import jax
import jax.numpy as jnp
from jax import lax
from jax.experimental import pallas as pl
from jax.experimental.pallas import tpu as pltpu
from jax.experimental.pallas import tpu_sc as plsc

_B = 4096
_D0, _D1, _D2 = 128, 64, 32
_NC, _NS, _L = 2, 16, 16
_NW = _NC * _NS
_BPW = _B // _NW


def _sc_body(idx_hbm, t0_hbm, t1_hbm, t2_hbm, out_hbm,
             idx_v, b0, b1, b2, sem0, sem1, sem2):
    wid = lax.axis_index("s") * _NC + lax.axis_index("c")
    base = wid * _BPW
    pltpu.sync_copy(idx_hbm.at[pl.ds(base, _BPW)], idx_v)
    c0 = pltpu.async_copy(t0_hbm.at[idx_v], b0, sem0)
    lane = lax.iota(jnp.int32, _L)

    for j in range(_BPW // _L):
        iv = idx_v[pl.ds(_L * j, _L)]
        for l in range(_L):
            row = jnp.sum(jnp.where(lane == l, iv, 0))
            r = _L * j + l
            pltpu.async_copy(t1_hbm.at[pl.ds(row, 1)], b1.at[pl.ds(r, 1)], sem1)
            pltpu.async_copy(t2_hbm.at[pl.ds(row, 1)], b2.at[pl.ds(r, 1)], sem2)

    pltpu.make_async_copy(t1_hbm.at[pl.ds(0, _BPW)], b1, sem1).wait()
    pltpu.make_async_copy(t2_hbm.at[pl.ds(0, _BPW)], b2, sem2).wait()
    c0.wait()

    third = jnp.float32(1.0 / 3.0)

    def row_body(r, carry):
        for j in range(_D0 // _L):
            v = b0[r, pl.ds(_L * j, _L)]
            if _L * j < _D1:
                v = v + b1[r, pl.ds(_L * j, _L)]
            if _L * j < _D2:
                v = v + b2[r, pl.ds(_L * j, _L)]
            b0[r, pl.ds(_L * j, _L)] = v * third
        return carry

    lax.fori_loop(0, _BPW, row_body, 0)
    pltpu.sync_copy(b0, out_hbm.at[pl.ds(base, _BPW)])


def kernel(indexes, table0, table1, table2):
    mesh = plsc.VectorSubcoreMesh(core_axis_name="c", subcore_axis_name="s")
    k = pl.kernel(
        _sc_body,
        out_type=jax.ShapeDtypeStruct((_B, _D0), jnp.float32),
        mesh=mesh,
        compiler_params=pltpu.CompilerParams(needs_layout_passes=False),
        scratch_types=[
            pltpu.VMEM((_BPW,), jnp.int32),
            pltpu.VMEM((_BPW, _D0), jnp.float32),
            pltpu.VMEM((_BPW, _D1), jnp.float32),
            pltpu.VMEM((_BPW, _D2), jnp.float32),
            pltpu.SemaphoreType.DMA,
            pltpu.SemaphoreType.DMA,
            pltpu.SemaphoreType.DMA,
        ],
    )
    return k(indexes.astype(jnp.int32), table0, table1, table2)


# speedup vs baseline: 2.4523x; 1.6847x over previous
import jax
import jax.numpy as jnp
from jax import lax
from jax.experimental import pallas as pl
from jax.experimental.pallas import tpu as pltpu
from jax.experimental.pallas import tpu_sc as plsc

_B = 4096
_E = 100000
_D0, _D1, _D2 = 128, 64, 32
_NF = _D1 + _D2
_NC, _NS, _L = 2, 16, 16
_NW = _NC * _NS
_BPW = _B // _NW
_FPW = _NF // _NW


def _sc_body(idx_hbm, t0_hbm, t1t_hbm, t2t_hbm, part0_hbm, outt_hbm,
             idx_v, idx_all, fbuf, fval, b0, sem0, semf):
    wid = lax.axis_index("s") * _NC + lax.axis_index("c")
    base = wid * _BPW
    pltpu.sync_copy(idx_hbm.at[pl.ds(base, _BPW)], idx_v)
    c0 = pltpu.async_copy(t0_hbm.at[idx_v], b0, sem0)
    pltpu.sync_copy(idx_hbm, idx_all)

    for p in range(_FPW):
        f = wid * _FPW + p

        @pl.when(f < _D1)
        def _(f=f):
            pltpu.sync_copy(t1t_hbm.at[f], fbuf)

        @pl.when(f >= _D1)
        def _(f=f):
            pltpu.sync_copy(t2t_hbm.at[f - _D1], fbuf)

        def chunk(k, carry, f=f):
            iv = idx_all[pl.ds(_L * k, _L)]
            fval[pl.ds(_L * k, _L)] = plsc.load_gather(fbuf, [iv])
            return carry

        lax.fori_loop(0, _B // _L, chunk, 0)
        pltpu.sync_copy(fval, outt_hbm.at[f])

    c0.wait()
    pltpu.sync_copy(b0, part0_hbm.at[pl.ds(base, _BPW)])


def kernel(indexes, table0, table1, table2):
    t1t = jnp.transpose(table1)
    t2t = jnp.transpose(table2)
    mesh = plsc.VectorSubcoreMesh(core_axis_name="c", subcore_axis_name="s")
    k = pl.kernel(
        _sc_body,
        out_type=(
            jax.ShapeDtypeStruct((_B, _D0), jnp.float32),
            jax.ShapeDtypeStruct((_NF, _B), jnp.float32),
        ),
        mesh=mesh,
        compiler_params=pltpu.CompilerParams(needs_layout_passes=False),
        scratch_types=[
            pltpu.VMEM((_BPW,), jnp.int32),
            pltpu.VMEM((_B,), jnp.int32),
            pltpu.VMEM((_E,), jnp.float32),
            pltpu.VMEM((_B,), jnp.float32),
            pltpu.VMEM((_BPW, _D0), jnp.float32),
            pltpu.SemaphoreType.DMA,
            pltpu.SemaphoreType.DMA,
        ],
    )
    part0, outt = k(indexes.astype(jnp.int32), table0, t1t, t2t)
    p1 = jnp.transpose(outt[:_D1])
    p2 = jnp.transpose(outt[_D1:])
    third = jnp.float32(1.0 / 3.0)
    left = part0[:, :_D2] + p1[:, :_D2] + p2
    mid = part0[:, _D2:_D1] + p1[:, _D2:]
    right = part0[:, _D1:]
    return jnp.concatenate([left, mid, right], axis=1) * third


# speedup vs baseline: 2.7430x; 1.1186x over previous
import jax
import jax.numpy as jnp
from jax import lax
from jax.experimental import pallas as pl
from jax.experimental.pallas import tpu as pltpu
from jax.experimental.pallas import tpu_sc as plsc

_B = 4096
_E = 100000
_D0, _D1, _D2 = 128, 64, 32
_NC, _NS, _L = 2, 16, 16
_NW = _NC * _NS
_BPW = _B // _NW


def _sc_body(idx_hbm, t0_hbm, t1t_hbm, t2t_hbm, part0_hbm, outt_hbm,
             idx_v, idx_all, fbuf, fval, b0, sem0, semf):
    wid = lax.axis_index("s") * _NC + lax.axis_index("c")
    base = wid * _BPW
    pltpu.sync_copy(idx_hbm.at[pl.ds(base, _BPW)], idx_v)
    c0 = pltpu.async_copy(t0_hbm.at[idx_v], b0, sem0)
    pltpu.sync_copy(idx_hbm, idx_all)

    def gather_pass(accumulate):
        def chunk(k, carry):
            iv = idx_all[pl.ds(_L * k, _L)]
            g = plsc.load_gather(fbuf, [iv])
            if accumulate:
                g = g + fval[pl.ds(_L * k, _L)]
            fval[pl.ds(_L * k, _L)] = g
            return carry

        lax.fori_loop(0, _B // _L, chunk, 0)

    pltpu.sync_copy(t1t_hbm.at[wid], fbuf)
    gather_pass(False)
    pltpu.sync_copy(t2t_hbm.at[wid], fbuf)
    gather_pass(True)
    pltpu.sync_copy(fval, outt_hbm.at[wid])

    pltpu.sync_copy(t1t_hbm.at[_D2 + wid], fbuf)
    gather_pass(False)
    pltpu.sync_copy(fval, outt_hbm.at[_D2 + wid])

    c0.wait()
    pltpu.sync_copy(b0, part0_hbm.at[pl.ds(base, _BPW)])


def kernel(indexes, table0, table1, table2):
    t1t = jnp.transpose(table1)
    t2t = jnp.transpose(table2)
    mesh = plsc.VectorSubcoreMesh(core_axis_name="c", subcore_axis_name="s")
    k = pl.kernel(
        _sc_body,
        out_type=(
            jax.ShapeDtypeStruct((_B, _D0), jnp.float32),
            jax.ShapeDtypeStruct((_D1, _B), jnp.float32),
        ),
        mesh=mesh,
        compiler_params=pltpu.CompilerParams(needs_layout_passes=False),
        scratch_types=[
            pltpu.VMEM((_BPW,), jnp.int32),
            pltpu.VMEM((_B,), jnp.int32),
            pltpu.VMEM((_E,), jnp.float32),
            pltpu.VMEM((_B,), jnp.float32),
            pltpu.VMEM((_BPW, _D0), jnp.float32),
            pltpu.SemaphoreType.DMA,
            pltpu.SemaphoreType.DMA,
        ],
    )
    part0, outt = k(indexes.astype(jnp.int32), table0, t1t, t2t)
    third = jnp.float32(1.0 / 3.0)
    left = part0[:, :_D1] + jnp.transpose(outt)
    return jnp.concatenate([left, part0[:, _D1:]], axis=1) * third


# speedup vs baseline: 2.7681x; 1.0091x over previous
import jax
import jax.numpy as jnp
from jax import lax
from jax.experimental import pallas as pl
from jax.experimental.pallas import tpu as pltpu
from jax.experimental.pallas import tpu_sc as plsc

_B = 4096
_E = 100000
_D0, _D1, _D2 = 128, 64, 32
_NC, _NS, _L = 2, 16, 16
_NW = _NC * _NS
_BPW = _B // _NW


_SPLIT = 50176


def _sc_body(idx_hbm, t0_hbm, t1t_hbm, t2t_hbm, part0_hbm, outt_hbm,
             idx_v, idx_all, bufa, bufb, fval, b0, sem0, sema, semb):
    wid = lax.axis_index("s") * _NC + lax.axis_index("c")
    base = wid * _BPW
    pltpu.sync_copy(idx_hbm.at[pl.ds(base, _BPW)], idx_v)
    c0 = pltpu.async_copy(t0_hbm.at[idx_v], b0, sem0)
    pltpu.sync_copy(idx_hbm, idx_all)

    def fire(tab, f, half):
        if half == 0:
            return pltpu.async_copy(tab.at[f, pl.ds(0, _SPLIT)], bufa, sema)
        return pltpu.async_copy(tab.at[f, pl.ds(_SPLIT, _E - _SPLIT)], bufb, semb)

    def gather_half(half, accumulate):
        buf = bufa if half == 0 else bufb

        def chunk(k, carry):
            iv = idx_all[pl.ds(_L * k, _L)]
            if half == 0:
                m = iv < _SPLIT
            else:
                m = iv >= _SPLIT
                iv = iv - _SPLIT
            g = jnp.where(m, plsc.load_gather(buf, [iv], mask=m), 0.0)
            if accumulate:
                g = g + fval[pl.ds(_L * k, _L)]
            fval[pl.ds(_L * k, _L)] = g
            return carry

        lax.fori_loop(0, _B // _L, chunk, 0)

    rows = [
        (t1t_hbm, wid, False, None),
        (t2t_hbm, wid, True, wid),
        (t1t_hbm, _D2 + wid, False, _D2 + wid),
    ]
    ca = fire(*rows[0][:2], 0)
    cb = fire(*rows[0][:2], 1)
    for i, (tab, f, acc, wout) in enumerate(rows):
        ca.wait()
        gather_half(0, acc)
        if i + 1 < len(rows):
            ca = fire(*rows[i + 1][:2], 0)
        cb.wait()
        gather_half(1, True)
        if i + 1 < len(rows):
            cb = fire(*rows[i + 1][:2], 1)
        if wout is not None:
            pltpu.sync_copy(fval, outt_hbm.at[wout])

    c0.wait()
    pltpu.sync_copy(b0, part0_hbm.at[pl.ds(base, _BPW)])


def kernel(indexes, table0, table1, table2):
    t1t = jnp.transpose(table1)
    t2t = jnp.transpose(table2)
    mesh = plsc.VectorSubcoreMesh(core_axis_name="c", subcore_axis_name="s")
    k = pl.kernel(
        _sc_body,
        out_type=(
            jax.ShapeDtypeStruct((_B, _D0), jnp.float32),
            jax.ShapeDtypeStruct((_D1, _B), jnp.float32),
        ),
        mesh=mesh,
        compiler_params=pltpu.CompilerParams(needs_layout_passes=False),
        scratch_types=[
            pltpu.VMEM((_BPW,), jnp.int32),
            pltpu.VMEM((_B,), jnp.int32),
            pltpu.VMEM((_SPLIT,), jnp.float32),
            pltpu.VMEM((_E - _SPLIT,), jnp.float32),
            pltpu.VMEM((_B,), jnp.float32),
            pltpu.VMEM((_BPW, _D0), jnp.float32),
            pltpu.SemaphoreType.DMA,
            pltpu.SemaphoreType.DMA,
            pltpu.SemaphoreType.DMA,
        ],
    )
    part0, outt = k(indexes.astype(jnp.int32), table0, t1t, t2t)
    third = jnp.float32(1.0 / 3.0)
    left = part0[:, :_D1] + jnp.transpose(outt)
    return jnp.concatenate([left, part0[:, _D1:]], axis=1) * third
